# baseline (speedup 1.0000x reference)
import jax
import jax.numpy as jnp
from jax import lax
from jax.experimental import pallas as pl
from jax.experimental.pallas import tpu as pltpu
from jax.experimental.pallas import tpu_sc as plsc

BATCH = 16384
EMBED_K = 16
NC = 2
NS = 16
L = 16
NW = NC * NS
B_PER_W = BATCH // NW
N_GROUPS = B_PER_W // L
IDX_CHUNK = 128
N_CHUNKS = B_PER_W // IDX_CHUNK


def _mf_body(x_hbm, w_hbm, h_hbm, out_hbm,
             xbuf, uidx, vidx, urows, vrows, outv, usem, vsem):
    wid = lax.axis_index("s") * NC + lax.axis_index("c")
    base = pl.multiple_of(wid * B_PER_W, B_PER_W)

    pltpu.sync_copy(x_hbm.at[pl.ds(base * 2, 2 * B_PER_W)], xbuf)

    lane = lax.iota(jnp.int32, L)

    def extract(g, _):
        pair = (jnp.full((L,), 2 * g * L, jnp.int32) + 2 * lane)
        off = pl.ds(pl.multiple_of(g * L, L), L)
        uidx[off] = plsc.load_gather(xbuf, [pair])
        vidx[off] = plsc.load_gather(xbuf, [pair + 1])
        return 0

    lax.fori_loop(0, N_GROUPS, extract, 0, unroll=4)

    copies = []
    for j in range(N_CHUNKS):
        sl = pl.ds(j * IDX_CHUNK, IDX_CHUNK)
        copies.append(pltpu.async_copy(
            w_hbm.at[uidx.at[sl]], urows.at[sl], usem))
        copies.append(pltpu.async_copy(
            h_hbm.at[vidx.at[sl]], vrows.at[sl], vsem))
    for cp in copies:
        cp.wait()

    def compute(g, _):
        row = jnp.full((L,), g * L, jnp.int32) + lane
        acc = jnp.zeros((L,), jnp.float32)
        for k in range(EMBED_K):
            ck = jnp.full((L,), k, jnp.int32)
            acc += plsc.load_gather(urows, [row, ck]) * \
                   plsc.load_gather(vrows, [row, ck])
        outv[pl.ds(pl.multiple_of(g * L, L), L)] = acc
        return 0

    lax.fori_loop(0, N_GROUPS, compute, 0, unroll=2)

    pltpu.sync_copy(outv, out_hbm.at[pl.ds(base, B_PER_W)])


@jax.jit
def _mf_kernel(x, W, H):
    mesh = plsc.VectorSubcoreMesh(core_axis_name="c", subcore_axis_name="s")
    return pl.kernel(
        _mf_body,
        out_type=jax.ShapeDtypeStruct((BATCH,), jnp.float32),
        mesh=mesh,
        compiler_params=pltpu.CompilerParams(
            needs_layout_passes=False, use_tc_tiling_on_sc=False),
        scratch_types=[
            pltpu.VMEM((2 * B_PER_W,), jnp.int32),
            pltpu.VMEM((B_PER_W,), jnp.int32),
            pltpu.VMEM((B_PER_W,), jnp.int32),
            pltpu.VMEM((B_PER_W, EMBED_K), jnp.float32),
            pltpu.VMEM((B_PER_W, EMBED_K), jnp.float32),
            pltpu.VMEM((B_PER_W,), jnp.float32),
            pltpu.SemaphoreType.DMA,
            pltpu.SemaphoreType.DMA,
        ],
    )(x, W, H)


def kernel(x, W, H):
    return _mf_kernel(x.reshape(-1), W, H)


# speedup vs baseline: 1.4526x; 1.4526x over previous
import jax
import jax.numpy as jnp
from jax import lax
from jax.experimental import pallas as pl
from jax.experimental.pallas import tpu as pltpu
from jax.experimental.pallas import tpu_sc as plsc

BATCH = 16384
EMBED_K = 16
NC = 2
NS = 16
L = 16
NW = NC * NS
B_PER_W = BATCH // NW
CH = 256
N_CHUNKS = B_PER_W // CH


def _mf_body(x_hbm, w_hbm, h_hbm, out_hbm,
             xbuf, urows, vrows, outv, usem, vsem):
    wid = lax.axis_index("s") * NC + lax.axis_index("c")
    base = pl.multiple_of(wid * B_PER_W, B_PER_W)

    pltpu.sync_copy(x_hbm.at[pl.ds(base * 2, 2 * B_PER_W)], xbuf)

    lane = lax.iota(jnp.int32, L)

    for c in range(N_CHUNKS):
        def fire(g, _):
            off = pl.multiple_of(c * 2 * CH + g * L, L)
            pairs = xbuf[pl.ds(off, L)]
            for j in range(L // 2):
                b = g * (L // 2) + j
                pltpu.async_copy(w_hbm.at[pl.ds(pairs[2 * j], 1)],
                                 urows.at[pl.ds(b, 1)], usem)
                pltpu.async_copy(h_hbm.at[pl.ds(pairs[2 * j + 1], 1)],
                                 vrows.at[pl.ds(b, 1)], vsem)
            return 0

        lax.fori_loop(0, 2 * CH // L, fire, 0)

        pltpu.make_async_copy(w_hbm.at[pl.ds(0, CH)], urows, usem).wait()
        pltpu.make_async_copy(h_hbm.at[pl.ds(0, CH)], vrows, vsem).wait()

        def compute(g, _):
            row = jnp.full((L,), g * L, jnp.int32) + lane
            acc = jnp.zeros((L,), jnp.float32)
            for k in range(EMBED_K):
                ck = jnp.full((L,), k, jnp.int32)
                acc += plsc.load_gather(urows, [row, ck]) * \
                       plsc.load_gather(vrows, [row, ck])
            outv[pl.ds(pl.multiple_of(g * L, L), L)] = acc
            return 0

        lax.fori_loop(0, CH // L, compute, 0, unroll=2)

        pltpu.sync_copy(outv, out_hbm.at[pl.ds(base + c * CH, CH)])


@jax.jit
def _mf_kernel(x, W, H):
    mesh = plsc.VectorSubcoreMesh(core_axis_name="c", subcore_axis_name="s")
    return pl.kernel(
        _mf_body,
        out_type=jax.ShapeDtypeStruct((BATCH,), jnp.float32),
        mesh=mesh,
        compiler_params=pltpu.CompilerParams(needs_layout_passes=False),
        scratch_types=[
            pltpu.VMEM((2 * B_PER_W,), jnp.int32),
            pltpu.VMEM((CH, EMBED_K), jnp.float32),
            pltpu.VMEM((CH, EMBED_K), jnp.float32),
            pltpu.VMEM((CH,), jnp.float32),
            pltpu.SemaphoreType.DMA,
            pltpu.SemaphoreType.DMA,
        ],
    )(x, W, H)


def kernel(x, W, H):
    return _mf_kernel(x.reshape(-1), W, H)
